# baseline (speedup 1.0000x reference)
import functools

import jax
import jax.numpy as jnp
from jax import lax
from jax.experimental import pallas as pl
from jax.experimental.pallas import tpu as pltpu
from jax.experimental.pallas import tpu_sc as plsc

D_MODEL = 1024
BATCH = 4
SEQ = 4096

NC = 2
NS = 16
NW = NC * NS
S_PER_W = SEQ // NW
CH = 32
LANES = 16


def _body(x_hbm, pe_hbm, table_hbm, out_hbm, idx_v, pe_v, rows_v, sem):
    wid = lax.axis_index("s") * NC + lax.axis_index("c")
    s_base = wid * S_PER_W

    def chunk_body(j, carry):
        s0 = s_base + j * CH
        pltpu.sync_copy(pe_hbm.at[pl.ds(s0, CH)], pe_v)

        def batch_body(b, carry2):
            pltpu.sync_copy(x_hbm.at[b, pl.ds(s0, CH)], idx_v)
            pltpu.async_copy(table_hbm.at[idx_v], rows_v, sem).wait()

            def row_body(r, carry3):
                for q in range(D_MODEL // LANES):
                    sl = pl.ds(q * LANES, LANES)
                    rows_v[r, sl] = rows_v[r, sl] + pe_v[r, sl]
                return carry3

            lax.fori_loop(0, CH, row_body, 0)
            pltpu.sync_copy(rows_v, out_hbm.at[b, pl.ds(s0, CH)])
            return carry2

        lax.fori_loop(0, BATCH, batch_body, 0)
        return carry

    lax.fori_loop(0, S_PER_W // CH, chunk_body, 0)


@functools.partial(jax.jit, static_argnames=())
def kernel(x, table, pe):
    xi = x.astype(jnp.int32)
    pe2 = pe.reshape(pe.shape[1], pe.shape[2])
    mesh = plsc.VectorSubcoreMesh(core_axis_name="c", subcore_axis_name="s")
    run = functools.partial(
        pl.kernel,
        out_type=jax.ShapeDtypeStruct((BATCH, SEQ, D_MODEL), jnp.float32),
        mesh=mesh,
        scratch_types=[
            pltpu.VMEM((CH,), jnp.int32),
            pltpu.VMEM((CH, D_MODEL), jnp.float32),
            pltpu.VMEM((CH, D_MODEL), jnp.float32),
            pltpu.SemaphoreType.DMA,
        ],
    )(_body)
    return run(xi, pe2, table)
